# baseline (speedup 1.0000x reference)
import functools

import jax
import jax.numpy as jnp
from jax.experimental import pallas as pl
from jax.experimental.pallas import tpu as pltpu


def _fused_kernel(xin_ref, w_ref, bias_ref, smp_ref, p_ref, out_ref,
                  xt_ref, *, B, F, n, blkN):
    i = pl.program_id(0)

    @pl.when(i == 0)
    def _linear():
        wt = w_ref[...].T.astype(jnp.bfloat16)
        for bb in range(B):
            y = jnp.dot(xin_ref[bb * n:(bb + 1) * n, :], wt,
                        preferred_element_type=jnp.float32)
            xt_ref[:, bb * F:(bb + 1) * F] = (
                y + bias_ref[...]).astype(jnp.bfloat16)

    p = p_ref[...].astype(jnp.bfloat16)
    rows = i * blkN + jax.lax.broadcasted_iota(jnp.int32, (blkN, n), 0)
    s_onehot = (smp_ref[...] == rows)
    rowmask = jnp.any(s_onehot, axis=1, keepdims=True)
    p_eff = jnp.where(rowmask, s_onehot.astype(jnp.bfloat16), p)
    out_ref[...] = jnp.dot(p_eff, xt_ref[...],
                           preferred_element_type=jnp.float32)


def kernel(input, P, sample, W, b):
    Bz, n, F = input.shape
    N = P.shape[0]
    BF = Bz * F
    blkN = 512

    xin = input.astype(jnp.bfloat16).reshape(Bz * n, F)

    y2 = pl.pallas_call(
        functools.partial(_fused_kernel, B=Bz, F=F, n=n, blkN=blkN),
        grid=(N // blkN,),
        in_specs=[
            pl.BlockSpec((Bz * n, F), lambda i: (0, 0)),
            pl.BlockSpec((F, F), lambda i: (0, 0)),
            pl.BlockSpec((1, F), lambda i: (0, 0)),
            pl.BlockSpec((1, n), lambda i: (0, 0)),
            pl.BlockSpec((blkN, n), lambda i: (i, 0)),
        ],
        out_specs=pl.BlockSpec((blkN, BF), lambda i: (i, 0)),
        out_shape=jax.ShapeDtypeStruct((N, BF), jnp.float32),
        scratch_shapes=[pltpu.VMEM((n, BF), jnp.bfloat16)],
    )(xin, W, b.reshape(1, F), sample.reshape(1, n).astype(jnp.int32), P)

    return y2.reshape(N, Bz, F).transpose(1, 0, 2)


# speedup vs baseline: 1.0420x; 1.0420x over previous
import functools

import jax
import jax.numpy as jnp
from jax.experimental import pallas as pl
from jax.experimental.pallas import tpu as pltpu


def _linear_kernel(xin_ref, w_ref, bias_ref, xt_ref, *, B, F):
    wt = w_ref[...].T.astype(jnp.bfloat16)
    for bb in range(B):
        sl = slice(bb * F, (bb + 1) * F)
        y = jnp.dot(xin_ref[:, sl], wt, preferred_element_type=jnp.float32)
        xt_ref[:, sl] = (y + bias_ref[...]).astype(jnp.bfloat16)


def _matmul_kernel(xt_ref, smp_ref, p_ref, out_ref, *, n, blkN):
    i = pl.program_id(0)
    p = p_ref[...].astype(jnp.bfloat16)
    rows = i * blkN + jax.lax.broadcasted_iota(jnp.int32, (blkN, n), 0)
    s_onehot = (smp_ref[...] == rows)
    rowmask = jnp.any(s_onehot, axis=1, keepdims=True)
    p_eff = jnp.where(rowmask, s_onehot.astype(jnp.bfloat16), p)
    out_ref[...] = jnp.dot(p_eff, xt_ref[...],
                           preferred_element_type=jnp.float32)


def kernel(input, P, sample, W, b):
    Bz, n, F = input.shape
    N = P.shape[0]
    BF = Bz * F
    blkN = 512

    xin = input.astype(jnp.bfloat16).transpose(1, 0, 2).reshape(n, BF)

    xt = pl.pallas_call(
        functools.partial(_linear_kernel, B=Bz, F=F),
        in_specs=[
            pl.BlockSpec((n, BF), lambda: (0, 0)),
            pl.BlockSpec((F, F), lambda: (0, 0)),
            pl.BlockSpec((1, F), lambda: (0, 0)),
        ],
        out_specs=pl.BlockSpec((n, BF), lambda: (0, 0)),
        out_shape=jax.ShapeDtypeStruct((n, BF), jnp.bfloat16),
    )(xin, W, b.reshape(1, F))

    y2 = pl.pallas_call(
        functools.partial(_matmul_kernel, n=n, blkN=blkN),
        grid=(N // blkN,),
        in_specs=[
            pl.BlockSpec((n, BF), lambda i: (0, 0)),
            pl.BlockSpec((1, n), lambda i: (0, 0)),
            pl.BlockSpec((blkN, n), lambda i: (i, 0)),
        ],
        out_specs=pl.BlockSpec((blkN, BF), lambda i: (i, 0)),
        out_shape=jax.ShapeDtypeStruct((N, BF), jnp.float32),
    )(xt, sample.reshape(1, n).astype(jnp.int32), P)

    return y2.reshape(N, Bz, F).transpose(1, 0, 2)


# speedup vs baseline: 1.0500x; 1.0077x over previous
import functools

import jax
import jax.numpy as jnp
from jax.experimental import pallas as pl
from jax.experimental.pallas import tpu as pltpu


def _linear_kernel(xin_ref, w_ref, bias_ref, xt_ref, *, B, F):
    wt = w_ref[...].T
    for bb in range(B):
        sl = slice(bb * F, (bb + 1) * F)
        y = jnp.dot(xin_ref[:, sl], wt, preferred_element_type=jnp.float32)
        xt_ref[:, sl] = (y + bias_ref[...]).astype(jnp.bfloat16)


def _matmul_kernel(xt_ref, smp_ref, p_ref, out_ref, *, n, blkN):
    i = pl.program_id(0)
    p = p_ref[...].astype(jnp.bfloat16)
    rows = i * blkN + jax.lax.broadcasted_iota(jnp.int32, (blkN, n), 0)
    s_onehot = (smp_ref[...] == rows)
    rowmask = jnp.any(s_onehot, axis=1, keepdims=True)
    p_eff = jnp.where(rowmask, s_onehot.astype(jnp.bfloat16), p)
    out_ref[...] = jnp.dot(p_eff, xt_ref[...],
                           preferred_element_type=jnp.float32)


def kernel(input, P, sample, W, b):
    Bz, n, F = input.shape
    N = P.shape[0]
    BF = Bz * F
    blkN = 512

    xin = input.transpose(1, 0, 2).reshape(n, BF)

    xt = pl.pallas_call(
        functools.partial(_linear_kernel, B=Bz, F=F),
        in_specs=[
            pl.BlockSpec((n, BF), lambda: (0, 0)),
            pl.BlockSpec((F, F), lambda: (0, 0)),
            pl.BlockSpec((1, F), lambda: (0, 0)),
        ],
        out_specs=pl.BlockSpec((n, BF), lambda: (0, 0)),
        out_shape=jax.ShapeDtypeStruct((n, BF), jnp.bfloat16),
    )(xin, W, b.reshape(1, F))

    y2 = pl.pallas_call(
        functools.partial(_matmul_kernel, n=n, blkN=blkN),
        grid=(N // blkN,),
        in_specs=[
            pl.BlockSpec((n, BF), lambda i: (0, 0)),
            pl.BlockSpec((1, n), lambda i: (0, 0)),
            pl.BlockSpec((blkN, n), lambda i: (i, 0)),
        ],
        out_specs=pl.BlockSpec((blkN, BF), lambda i: (i, 0)),
        out_shape=jax.ShapeDtypeStruct((N, BF), jnp.float32),
    )(xt, sample.reshape(1, n).astype(jnp.int32), P)

    return y2.reshape(N, Bz, F).transpose(1, 0, 2)
